# baseline (speedup 1.0000x reference)
import functools

import jax
import jax.numpy as jnp
from jax.experimental import pallas as pl
from jax.experimental.pallas import tpu as pltpu


def _make_kernel(nbuf, bm, num_blocks):
    def _gcn_kernel(a_ref, x_ref, w_ref, o_ref, bufs, y_ref, oacc, sems, osem):
        x = x_ref[...].astype(jnp.bfloat16)
        w = w_ref[...].astype(jnp.bfloat16)
        y_ref[...] = jnp.dot(x, w, preferred_element_type=jnp.float32).astype(
            jnp.bfloat16
        )

        def _start(step, slot):
            pltpu.make_async_copy(
                a_ref.at[pl.ds(step * bm, bm), :],
                bufs.at[slot],
                sems.at[slot],
            ).start()

        for j in range(nbuf):
            _start(j, j)

        y = y_ref[...]

        def body(i, _):
            slot = jax.lax.rem(i, nbuf)
            pltpu.make_async_copy(
                a_ref.at[pl.ds(i * bm, bm), :],
                bufs.at[slot],
                sems.at[slot],
            ).wait()
            a = bufs[slot].astype(jnp.bfloat16)
            oacc[pl.ds(i * bm, bm), :] = jnp.dot(
                a, y, preferred_element_type=jnp.float32
            )

            @pl.when(i + nbuf < num_blocks)
            def _():
                _start(i + nbuf, slot)

            return 0

        jax.lax.fori_loop(0, num_blocks, body, 0)
        cp = pltpu.make_async_copy(oacc, o_ref, osem)
        cp.start()
        cp.wait()

    return _gcn_kernel


@functools.partial(jax.jit, static_argnames=("bm", "nbuf"))
def _one_support(x, adj, w, bm, nbuf):
    n, in_f = x.shape
    out_f = w.shape[1]
    num_blocks = n // bm
    return pl.pallas_call(
        _make_kernel(nbuf, bm, num_blocks),
        in_specs=[
            pl.BlockSpec(memory_space=pl.ANY),
            pl.BlockSpec(memory_space=pltpu.VMEM),
            pl.BlockSpec(memory_space=pltpu.VMEM),
        ],
        out_specs=pl.BlockSpec(memory_space=pl.ANY),
        out_shape=jax.ShapeDtypeStruct((n, out_f), jnp.float32),
        scratch_shapes=[
            pltpu.VMEM((nbuf, bm, n), jnp.float32),
            pltpu.VMEM((n, out_f), jnp.bfloat16),
            pltpu.VMEM((n, out_f), jnp.float32),
            pltpu.SemaphoreType.DMA((nbuf,)),
            pltpu.SemaphoreType.DMA,
        ],
    )(adj, x, w)


def kernel(input, support, W):
    x = input
    out = None
    for i in range(support.shape[0]):
        o = _one_support(x, support[i], W[i], bm=200, nbuf=4)
        out = o if out is None else out + o
    return out


# speedup vs baseline: 1.0207x; 1.0207x over previous
import functools

import jax
import jax.numpy as jnp
from jax.experimental import pallas as pl
from jax.experimental.pallas import tpu as pltpu


def _gcn_kernel(a_ref, x_ref, w_ref, o_ref, y_ref):
    @pl.when(pl.program_id(0) == 0)
    def _compute_y():
        x = x_ref[...].astype(jnp.bfloat16)
        w = w_ref[...].astype(jnp.bfloat16)
        y_ref[...] = jnp.dot(x, w, preferred_element_type=jnp.float32).astype(
            jnp.bfloat16
        )

    a = a_ref[...].astype(jnp.bfloat16)
    o_ref[...] = jnp.dot(a, y_ref[...], preferred_element_type=jnp.float32)


@functools.partial(jax.jit, static_argnames=("bm",))
def _one_support(x, adj, w, bm):
    n, in_f = x.shape
    out_f = w.shape[1]
    num_blocks = pl.cdiv(n, bm)
    return pl.pallas_call(
        _gcn_kernel,
        grid=(num_blocks,),
        in_specs=[
            pl.BlockSpec((bm, n), lambda m: (m, 0)),
            pl.BlockSpec((n, in_f), lambda m: (0, 0)),
            pl.BlockSpec((in_f, out_f), lambda m: (0, 0)),
        ],
        out_specs=pl.BlockSpec((bm, out_f), lambda m: (m, 0)),
        out_shape=jax.ShapeDtypeStruct((n, out_f), jnp.float32),
        scratch_shapes=[pltpu.VMEM((n, out_f), jnp.bfloat16)],
        compiler_params=pltpu.CompilerParams(
            dimension_semantics=("arbitrary",),
        ),
    )(adj, x, w)


def kernel(input, support, W):
    x = input
    out = None
    for i in range(support.shape[0]):
        o = _one_support(x, support[i], W[i], bm=400)
        out = o if out is None else out + o
    return out
